# Initial kernel scaffold; baseline (speedup 1.0000x reference)
import jax
import jax.numpy as jnp
from jax.experimental import pallas as pl


def kernel(user_id, timestamp, user_table, ts_table, boundaries, ts_mean, ts_var):
    raise NotImplementedError("write your pallas kernel here")


# speedup vs baseline: 2.0220x; 2.0220x over previous
import functools

import jax
import jax.numpy as jnp
from jax import lax
from jax.experimental import pallas as pl
from jax.experimental.pallas import tpu as pltpu
from jax.experimental.pallas import tpu_sc as plsc

BATCH = 16384
DIM = 32
NBND_PAD = 1024
LANES = 16

_NC, _NS = 2, 16
NW = _NC * _NS
B_PER_W = BATCH // NW
N_CHUNK = B_PER_W // LANES


def _sc_body(uid_hbm, ts_hbm, utab_hbm, ttab_hbm, bnd_hbm, norm_hbm, out_hbm,
             idx_v, ts_v, bnd_v, bkt_v, tn_v, u_rows, t_rows, norm_v,
             sem_u, sem_t):
    wid = lax.axis_index("s") * _NC + lax.axis_index("c")
    base = wid * B_PER_W

    pltpu.sync_copy(uid_hbm.at[pl.ds(base, B_PER_W)], idx_v)
    gather_u = pltpu.async_copy(utab_hbm.at[idx_v], u_rows, sem_u)

    pltpu.sync_copy(ts_hbm.at[pl.ds(base, B_PER_W)], ts_v)
    pltpu.sync_copy(bnd_hbm, bnd_v)
    pltpu.sync_copy(norm_hbm, norm_v)
    mean = norm_v[pl.ds(0, LANES)]
    denom = norm_v[pl.ds(LANES, LANES)]

    def chunk_body(i, carry):
        t = ts_v[pl.ds(i * LANES, LANES)]
        pos = jnp.zeros((LANES,), jnp.int32)
        step = NBND_PAD // 2
        while step >= 1:
            probe = pos + (step - 1)
            val = plsc.load_gather(bnd_v, [probe])
            pos = jnp.where(val <= t, pos + step, pos)
            step //= 2
        bkt_v[pl.ds(i * LANES, LANES)] = pos
        rows = i * LANES + lax.iota(jnp.int32, LANES)
        plsc.store_scatter(tn_v, [rows, jnp.zeros((LANES,), jnp.int32)],
                           (t - mean) / denom)
        return carry

    lax.fori_loop(0, N_CHUNK, chunk_body, 0)

    gather_t = pltpu.async_copy(ttab_hbm.at[bkt_v], t_rows, sem_t)
    pltpu.sync_copy(tn_v, out_hbm.at[pl.ds(base, B_PER_W), pl.ds(2 * DIM, 1)])
    gather_u.wait()
    pltpu.sync_copy(u_rows, out_hbm.at[pl.ds(base, B_PER_W), pl.ds(0, DIM)])
    gather_t.wait()
    pltpu.sync_copy(t_rows, out_hbm.at[pl.ds(base, B_PER_W), pl.ds(DIM, DIM)])


@jax.jit
def kernel(user_id, timestamp, user_table, ts_table, boundaries, ts_mean,
           ts_var):
    idx = user_id.astype(jnp.int32)
    bnd_pad = jnp.full((NBND_PAD,), jnp.inf, jnp.float32).at[
        : boundaries.shape[0]].set(boundaries)
    norm = jnp.concatenate([
        jnp.full((LANES,), ts_mean, jnp.float32),
        jnp.full((LANES,), jnp.sqrt(ts_var + 1e-6), jnp.float32),
    ])

    mesh = plsc.VectorSubcoreMesh(core_axis_name="c", subcore_axis_name="s")
    run = pl.kernel(
        _sc_body,
        out_type=jax.ShapeDtypeStruct((BATCH, 2 * DIM + 1), jnp.float32),
        mesh=mesh,
        scratch_types=[
            pltpu.VMEM((B_PER_W,), jnp.int32),
            pltpu.VMEM((B_PER_W,), jnp.float32),
            pltpu.VMEM((NBND_PAD,), jnp.float32),
            pltpu.VMEM((B_PER_W,), jnp.int32),
            pltpu.VMEM((B_PER_W, 1), jnp.float32),
            pltpu.VMEM((B_PER_W, DIM), jnp.float32),
            pltpu.VMEM((B_PER_W, DIM), jnp.float32),
            pltpu.VMEM((2 * LANES,), jnp.float32),
            pltpu.SemaphoreType.DMA,
            pltpu.SemaphoreType.DMA,
        ],
        compiler_params=pltpu.CompilerParams(use_tc_tiling_on_sc=False,
                                             needs_layout_passes=False),
    )
    return run(idx, timestamp, user_table, ts_table, bnd_pad, norm)
